# baseline (speedup 1.0000x reference)
import functools
import jax
import jax.numpy as jnp
from jax import lax
from jax.experimental import pallas as pl
from jax.experimental.pallas import tpu as pltpu
from jax.experimental.pallas import tpu_sc as plsc

N_CORES = 10000
N_QUBITS = 100000
CORE_CAP = 32
N_EDGES = 160000
D = 256
B = 4

NW = 32
CPT = 320
CHUNK = 2000
NSCAN = CHUNK // 16
NCHUNK = N_QUBITS // CHUNK
NECHUNK_ALL = N_EDGES // CHUNK
GROWS = 32

HALF = N_CORES // 2
EPT = N_EDGES // 16
NECHUNK = EPT // CHUNK
TRASH_SH = HALF + 8

_NEG_INF = float("-inf")


def _prefix16(x, lane):
    for k in (1, 2, 4, 8):
        idx = jnp.maximum(lane - k, 0)
        x = x + x.at[idx].get(mode="promise_in_bounds") * (lane >= k).astype(jnp.int32)
    return x


def _rsqrt16(x):
    i = plsc.bitcast(x, jnp.int32)
    i = 0x5F3759DF - lax.shift_right_logical(i, 1)
    y = plsc.bitcast(i, jnp.float32)
    for _ in range(4):
        y = y * (1.5 - 0.5 * x * y * y)
    return y


def _stage1_body(alloc_hbm, qe_hbm, dummy_hbm, dst_hbm, w_hbm,
                 out_hbm, dinv_hbm,
                 acc, rows, rows_b, achunk, wchunk, qidx, dloc, wlist, dinv_buf,
                 counts, deg_s, dummy_v, sem, sem_b):
    cid = lax.axis_index("c")
    sid = lax.axis_index("s")
    wid = sid * 2 + cid
    base = pl.multiple_of(wid * CPT, 8)
    npc = jnp.minimum(CPT, N_CORES - base)

    pltpu.sync_copy(dummy_hbm, dummy_v)

    def _zi(i, _):
        qidx[pl.ds(i * 16, 16)] = jnp.zeros((16,), jnp.int32)
        return 0
    lax.fori_loop(0, (CHUNK + 2 * GROWS) // 16, _zi, 0)

    trash16 = jnp.full((16,), CPT, jnp.int32)
    neg16 = jnp.full((16,), _NEG_INF, jnp.float32)
    lane = lax.iota(jnp.int32, 16)

    def _dz(i, _):
        deg_s[i] = jnp.float32(1.0)
        return 0
    lax.fori_loop(0, CPT + 1, _dz, 0)

    def _dchunk(ch, _):
        pos = ch * CHUNK
        pltpu.sync_copy(dst_hbm.at[pl.ds(pos, CHUNK)], achunk)
        pltpu.sync_copy(w_hbm.at[pl.ds(pos, CHUNK)], wchunk)

        def _rt(i, _):
            dloc[pl.ds(i * 16, 16)] = trash16
            return 0
        lax.fori_loop(0, (CHUNK + 16) // 16, _rt, 0)

        def _dscan(i, nq):
            dv = achunk[pl.ds(i * 16, 16)]
            wv = wchunk[pl.ds(i * 16, 16)]
            m = (dv >= base) & (dv < base + npc)
            mi = m.astype(jnp.int32)
            incl = _prefix16(mi, lane)
            dest = nq + incl - mi
            plsc.store_scatter(dloc, [dest], dv - base, mask=m)
            plsc.store_scatter(wlist, [dest], wv, mask=m)
            return nq + incl[15]
        nq = lax.fori_loop(0, NSCAN, _dscan, jnp.int32(0))

        def _acond(g):
            return g < nq

        def _abody(g):
            ga = pl.multiple_of(g, 16)
            cv = dloc[pl.ds(ga, 16)]
            wv = wlist[pl.ds(ga, 16)]
            for j in range(16):
                c = cv[j]
                deg_s[c] = deg_s[c] + wv[j]
            return g + 16
        lax.while_loop(_acond, _abody, jnp.int32(0))
        return 0
    lax.fori_loop(0, NECHUNK_ALL, _dchunk, 0)

    def _dv(r, _):
        for j in range(16):
            c = r * 16 + j
            sval = lax.broadcast_in_dim(deg_s[c], (16,), ())
            cvec = jnp.full((16,), c, jnp.int32)
            plsc.store_scatter(dinv_buf, [cvec], sval, mask=lane == j)
        x = dinv_buf[pl.ds(r * 16, 16)]
        dinv_buf[pl.ds(r * 16, 16)] = _rsqrt16(x)
        return 0
    lax.fori_loop(0, CPT // 16, _dv, 0)
    pltpu.sync_copy(dinv_buf.at[pl.ds(0, CPT)], dinv_hbm.at[pl.ds(base, CPT)])

    def _batch(b, _):
        def _ra(i, _):
            c = i // 16
            k = i % 16
            acc[c, pl.ds(k * 16, 16)] = neg16
            return 0
        lax.fori_loop(0, (CPT + 1) * 16, _ra, 0)

        def _rc(i, _):
            counts[i] = jnp.int32(0)
            return 0
        lax.fori_loop(0, CPT + 1, _rc, 0)

        def _chunk(ch, _):
            pos = ch * CHUNK
            pltpu.sync_copy(alloc_hbm.at[pl.ds(b * N_QUBITS + pos, CHUNK)], achunk)

            def _rt(i, _):
                dloc[pl.ds(i * 16, 16)] = trash16
                return 0
            lax.fori_loop(0, (CHUNK + 2 * GROWS) // 16, _rt, 0)

            def _scan(i, nq):
                v = achunk[pl.ds(i * 16, 16)]
                m = (v >= base) & (v < base + npc)
                mi = m.astype(jnp.int32)
                incl = _prefix16(mi, lane)
                dest = nq + incl - mi
                plsc.store_scatter(qidx, [dest], pos + i * 16 + lane, mask=m)
                plsc.store_scatter(dloc, [dest], v - base, mask=m)
                return nq + incl[15]
            nq = lax.fori_loop(0, NSCAN, _scan, jnp.int32(0))

            def _gcond(g):
                return g < nq

            def _proc(buf, ga):
                for g2 in range(0, GROWS, 16):
                    cv = dloc[pl.ds(pl.multiple_of(ga + g2, 16), 16)]
                    for j in range(16):
                        c = cv[j]
                        news = [jnp.maximum(acc[c, pl.ds(k * 16, 16)],
                                            buf[g2 + j, pl.ds(k * 16, 16)])
                                for k in range(16)]
                        for k in range(16):
                            acc[c, pl.ds(k * 16, 16)] = news[k]
                        counts[c] = counts[c] + 1

            def _gbody(g):
                ga = pl.multiple_of(g, 2 * GROWS)
                da = pltpu.async_copy(qe_hbm.at[qidx.at[pl.ds(ga, GROWS)]],
                                      rows, sem)
                db = pltpu.async_copy(
                    qe_hbm.at[qidx.at[pl.ds(ga + GROWS, GROWS)]], rows_b, sem_b)
                da.wait()
                _proc(rows, ga)
                db.wait()
                _proc(rows_b, ga + GROWS)
                return g + 2 * GROWS
            lax.while_loop(_gcond, _gbody, jnp.int32(0))
            return 0
        lax.fori_loop(0, NCHUNK, _chunk, 0)

        def _fin(c, _):
            madd = jnp.where(counts[c] < CORE_CAP, jnp.float32(0.0),
                             jnp.float32(_NEG_INF))
            mv = lax.broadcast_in_dim(madd, (16,), ())
            for k in range(16):
                sl = pl.ds(k * 16, 16)
                acc[c, sl] = jnp.maximum(acc[c, sl], dummy_v[sl] + mv)
            return 0
        lax.fori_loop(0, npc, _fin, 0)

        @pl.when(wid < NW - 1)
        def _():
            pltpu.sync_copy(acc.at[pl.ds(0, CPT)], out_hbm.at[b, pl.ds(base, CPT)])

        @pl.when(wid == NW - 1)
        def _():
            last = N_CORES - (NW - 1) * CPT
            pltpu.sync_copy(acc.at[pl.ds(0, last)], out_hbm.at[b, pl.ds(base, last)])
        return 0

    lax.fori_loop(0, B, _batch, 0)


_STAGE1_SCRATCH = [
    pltpu.VMEM((CPT + 1, D), jnp.float32),
    pltpu.VMEM((GROWS, D), jnp.float32),
    pltpu.VMEM((GROWS, D), jnp.float32),
    pltpu.VMEM((CHUNK,), jnp.int32),
    pltpu.VMEM((CHUNK,), jnp.float32),
    pltpu.VMEM((CHUNK + 2 * GROWS,), jnp.int32),
    pltpu.VMEM((CHUNK + 2 * GROWS,), jnp.int32),
    pltpu.VMEM((CHUNK + 16,), jnp.float32),
    pltpu.VMEM((CPT + 16,), jnp.float32),
    pltpu.SMEM((CPT + 1,), jnp.int32),
    pltpu.SMEM((CPT + 1,), jnp.float32),
    pltpu.VMEM((D,), jnp.float32),
    pltpu.SemaphoreType.DMA,
    pltpu.SemaphoreType.DMA,
]

_SC_MESH = plsc.VectorSubcoreMesh(core_axis_name="c", subcore_axis_name="s",
                                  num_cores=2, num_subcores=16)

_stage1 = functools.partial(
    pl.kernel,
    out_type=(jax.ShapeDtypeStruct((B, N_CORES, D), jnp.float32),
              jax.ShapeDtypeStruct((NW * CPT,), jnp.float32)),
    mesh=_SC_MESH,
    compiler_params=pltpu.CompilerParams(needs_layout_passes=False),
    scratch_types=_STAGE1_SCRATCH,
)(_stage1_body)


LB = 2048
CAP = 162688


def _route_body(src_hbm, dst_hbm, w_hbm, dinv_hbm,
                es_hbm, ec_hbm, en_hbm, ecnt_hbm,
                dinv_v, schunk, dchunk, wchunk, sidx, nrm, dloc, cnt_v, sem):
    cid = lax.axis_index("c")
    sid = lax.axis_index("s")
    wid = sid * 2 + cid
    base = pl.multiple_of(wid * CPT, 8)
    npc = jnp.minimum(CPT, N_CORES - base)
    lane = lax.iota(jnp.int32, 16)
    trash16 = jnp.full((16,), CPT, jnp.int32)

    pltpu.sync_copy(dinv_hbm, dinv_v)

    def _zi(i, _):
        sidx[pl.ds(i * 16, 16)] = jnp.zeros((16,), jnp.int32)
        nrm[pl.ds(i * 16, 16)] = jnp.zeros((16,), jnp.float32)
        return 0
    lax.fori_loop(0, LB // 16, _zi, 0)

    def _chunk(ch, off):
        epos = ch * CHUNK
        pltpu.sync_copy(src_hbm.at[pl.ds(epos, CHUNK)], schunk)
        pltpu.sync_copy(dst_hbm.at[pl.ds(epos, CHUNK)], dchunk)
        pltpu.sync_copy(w_hbm.at[pl.ds(epos, CHUNK)], wchunk)

        def _rt(i, _):
            dloc[pl.ds(i * 16, 16)] = trash16
            return 0
        lax.fori_loop(0, LB // 16, _rt, 0)

        def _scan(i, nq):
            sv = schunk[pl.ds(i * 16, 16)]
            dv = dchunk[pl.ds(i * 16, 16)]
            wv = wchunk[pl.ds(i * 16, 16)]
            m = (dv >= base) & (dv < base + npc)
            mi = m.astype(jnp.int32)
            incl = _prefix16(mi, lane)
            dest = nq + incl - mi
            nv = plsc.load_gather(dinv_v, [sv]) * wv * plsc.load_gather(dinv_v, [dv])
            plsc.store_scatter(sidx, [dest], sv, mask=m)
            plsc.store_scatter(nrm, [dest], nv, mask=m)
            plsc.store_scatter(dloc, [dest], dv - base, mask=m)
            return nq + incl[15]
        nq = lax.fori_loop(0, NSCAN, _scan, jnp.int32(0))

        offa = pl.multiple_of(off, 8)
        pltpu.sync_copy(sidx, es_hbm.at[pl.ds(wid * CAP + offa, LB)])
        pltpu.sync_copy(dloc, ec_hbm.at[pl.ds(wid * CAP + offa, LB)])
        pltpu.sync_copy(nrm, en_hbm.at[pl.ds(wid * CAP + offa, LB)])
        return off + ((nq + 7) // 8) * 8
    off = lax.fori_loop(0, NECHUNK_ALL, _chunk, jnp.int32(0))

    def _tb(i, _):
        sidx[pl.ds(i * 16, 16)] = jnp.zeros((16,), jnp.int32)
        nrm[pl.ds(i * 16, 16)] = jnp.zeros((16,), jnp.float32)
        dloc[pl.ds(i * 16, 16)] = trash16
        return 0
    lax.fori_loop(0, LB // 16, _tb, 0)
    offa = pl.multiple_of(off, 8)
    pltpu.sync_copy(sidx, es_hbm.at[pl.ds(wid * CAP + offa, LB)])
    pltpu.sync_copy(dloc, ec_hbm.at[pl.ds(wid * CAP + offa, LB)])
    pltpu.sync_copy(nrm, en_hbm.at[pl.ds(wid * CAP + offa, LB)])

    cnt_v[pl.ds(0, 16)] = lax.broadcast_in_dim(off, (16,), ())
    pltpu.sync_copy(cnt_v, ecnt_hbm.at[pl.ds(wid * 16, 16)])


_ROUTE_SCRATCH = [
    pltpu.VMEM((NW * CPT,), jnp.float32),
    pltpu.VMEM((CHUNK,), jnp.int32),
    pltpu.VMEM((CHUNK,), jnp.int32),
    pltpu.VMEM((CHUNK,), jnp.float32),
    pltpu.VMEM((LB,), jnp.int32),
    pltpu.VMEM((LB,), jnp.float32),
    pltpu.VMEM((LB,), jnp.int32),
    pltpu.VMEM((16,), jnp.int32),
    pltpu.SemaphoreType.DMA,
]

_route = functools.partial(
    pl.kernel,
    out_type=(jax.ShapeDtypeStruct((NW * CAP,), jnp.int32),
              jax.ShapeDtypeStruct((NW * CAP,), jnp.int32),
              jax.ShapeDtypeStruct((NW * CAP,), jnp.float32),
              jax.ShapeDtypeStruct((NW * 16,), jnp.int32)),
    mesh=_SC_MESH,
    compiler_params=pltpu.CompilerParams(needs_layout_passes=False),
    scratch_types=_ROUTE_SCRATCH,
)(_route_body)


def _msg_body(h_hbm, es_hbm, ec_hbm, en_hbm, ecnt_hbm, dinv_hbm, bias_hbm,
              y_hbm,
              dinv_v, lsrc, lcore, lnrm, gidx, gidx_b, acc, rows, rows_b,
              hrows, bias_v, cnt_v, sem, sem_b):
    cid = lax.axis_index("c")
    sid = lax.axis_index("s")
    wid = sid * 2 + cid
    base = pl.multiple_of(wid * CPT, 8)
    npc = jnp.minimum(CPT, N_CORES - base)
    lane = lax.iota(jnp.int32, 16)
    zero16 = jnp.zeros((16,), jnp.float32)

    pltpu.sync_copy(dinv_hbm, dinv_v)
    pltpu.sync_copy(bias_hbm, bias_v)
    pltpu.sync_copy(ecnt_hbm.at[pl.ds(wid * 16, 16)], cnt_v)
    ntot = cnt_v[pl.ds(0, 16)][0]

    def _batch(b, _):
        bN = b * N_CORES

        def _za(i, _):
            acc[i // 16, pl.ds((i % 16) * 16, 16)] = zero16
            return 0
        lax.fori_loop(0, (CPT + 1) * 16, _za, 0)

        def _bcond(o):
            return o < ntot

        def _bbody(o):
            oa = pl.multiple_of(o, 8)
            pltpu.sync_copy(es_hbm.at[pl.ds(wid * CAP + oa, LB)], lsrc)
            pltpu.sync_copy(ec_hbm.at[pl.ds(wid * CAP + oa, LB)], lcore)
            pltpu.sync_copy(en_hbm.at[pl.ds(wid * CAP + oa, LB)], lnrm)
            ng2 = (jnp.minimum(LB, ntot - oa) + 2 * GROWS - 1) // (2 * GROWS)

            def _fill_gidx(gref, ga):
                gref[pl.ds(0, 16)] = lsrc[pl.ds(pl.multiple_of(ga, 16), 16)] + bN
                gref[pl.ds(16, 16)] = lsrc[pl.ds(pl.multiple_of(ga + 16, 16), 16)] + bN

            def _proc(buf, ga):
                for g2 in range(0, GROWS, 16):
                    ga2 = pl.multiple_of(ga + g2, 16)
                    cv = lcore[pl.ds(ga2, 16)]
                    nvv = lnrm[pl.ds(ga2, 16)]
                    for j in range(16):
                        cvec = lax.broadcast_in_dim(cv[j], (16,), ())
                        sv16 = lax.broadcast_in_dim(nvv[j], (16,), ())
                        for k in range(16):
                            sl = pl.ds(k * 16, 16)
                            plsc.addupdate_scatter(
                                acc, [cvec, k * 16 + lane],
                                buf[g2 + j, sl] * sv16)

            def _group(gi, _):
                ga = gi * 2 * GROWS
                _fill_gidx(gidx, ga)
                da = pltpu.async_copy(h_hbm.at[gidx], rows, sem)
                _fill_gidx(gidx_b, ga + GROWS)
                db = pltpu.async_copy(h_hbm.at[gidx_b], rows_b, sem_b)
                da.wait()
                _proc(rows, ga)
                db.wait()
                _proc(rows_b, ga + GROWS)
                return 0
            lax.fori_loop(0, ng2, _group, 0)
            return o + LB
        lax.while_loop(_bcond, _bbody, jnp.int32(0))

        def _f(i, _):
            row0 = pl.multiple_of(i * 8, 8)
            grow0 = pl.multiple_of(bN + base + row0, 8)
            pltpu.sync_copy(h_hbm.at[pl.ds(grow0, 8)], hrows)
            dinvv = dinv_v[pl.ds(pl.multiple_of(base + row0, 8), 16)]
            for rr in range(8):
                ns = dinvv[rr]
                nsv = lax.broadcast_in_dim(ns * ns, (16,), ())
                for k in range(16):
                    sl = pl.ds(k * 16, 16)
                    y = acc[row0 + rr, sl] + hrows[rr, sl] * nsv + bias_v[sl]
                    acc[row0 + rr, sl] = jnp.maximum(y, 0.0)
            return 0
        lax.fori_loop(0, npc // 8, _f, 0)

        @pl.when(wid < NW - 1)
        def _():
            pltpu.sync_copy(acc.at[pl.ds(0, CPT)],
                            y_hbm.at[pl.ds(bN + base, CPT)])

        @pl.when(wid == NW - 1)
        def _():
            last = N_CORES - (NW - 1) * CPT
            pltpu.sync_copy(acc.at[pl.ds(0, last)],
                            y_hbm.at[pl.ds(bN + base, last)])
        return 0

    lax.fori_loop(0, B, _batch, 0)


_MSG_SCRATCH = [
    pltpu.VMEM((NW * CPT,), jnp.float32),
    pltpu.VMEM((LB,), jnp.int32),
    pltpu.VMEM((LB,), jnp.int32),
    pltpu.VMEM((LB,), jnp.float32),
    pltpu.VMEM((GROWS,), jnp.int32),
    pltpu.VMEM((GROWS,), jnp.int32),
    pltpu.VMEM((CPT + 1, D), jnp.float32),
    pltpu.VMEM((GROWS, D), jnp.float32),
    pltpu.VMEM((GROWS, D), jnp.float32),
    pltpu.VMEM((8, D), jnp.float32),
    pltpu.VMEM((D,), jnp.float32),
    pltpu.VMEM((16,), jnp.int32),
    pltpu.SemaphoreType.DMA,
    pltpu.SemaphoreType.DMA,
]

_msg = functools.partial(
    pl.kernel,
    out_type=jax.ShapeDtypeStruct((B * N_CORES, D), jnp.float32),
    mesh=_SC_MESH,
    compiler_params=pltpu.CompilerParams(needs_layout_passes=False),
    scratch_types=_MSG_SCRATCH,
)(_msg_body)


def _mm_body(x_ref, w_ref, o_ref):
    o_ref[...] = jnp.dot(x_ref[...], w_ref[...], preferred_element_type=jnp.float32)


def _matmul(x, w):
    rows = x.shape[0]
    blk = 400
    return pl.pallas_call(
        _mm_body,
        grid=(rows // blk,),
        in_specs=[
            pl.BlockSpec((blk, D), lambda i: (i, 0)),
            pl.BlockSpec((D, D), lambda i: (0, 0)),
        ],
        out_specs=pl.BlockSpec((blk, D), lambda i: (i, 0)),
        out_shape=jax.ShapeDtypeStruct((rows, D), jnp.float32),
    )(x, w)


def kernel(core_allocs, qubit_embs, dummy_qubit_emb, edge_index, edge_weight, W1, b1, W2, b2):
    src = edge_index[0]
    dst = edge_index[1]
    pre_embs, dinv = _stage1(core_allocs.reshape(-1), qubit_embs,
                             dummy_qubit_emb, dst, edge_weight)
    es, ec, en, ecnt = _route(src, dst, edge_weight, dinv)
    h1 = _matmul(pre_embs.reshape(B * N_CORES, D), W1)
    x1 = _msg(h1, es, ec, en, ecnt, dinv, b1)
    h2 = _matmul(x1, W2)
    x2 = _msg(h2, es, ec, en, ecnt, dinv, b2)
    return x2.reshape(B, N_CORES, D)


# speedup vs baseline: 1.5653x; 1.5653x over previous
import functools
import jax
import jax.numpy as jnp
from jax import lax
from jax.experimental import pallas as pl
from jax.experimental.pallas import tpu as pltpu
from jax.experimental.pallas import tpu_sc as plsc

N_CORES = 10000
N_QUBITS = 100000
CORE_CAP = 32
N_EDGES = 160000
D = 256
B = 4

NW = 32
CPT = 320
CHUNK = 2000
NSCAN = CHUNK // 16
NCHUNK = N_QUBITS // CHUNK
NECHUNK_ALL = N_EDGES // CHUNK
GROWS = 32

HALF = N_CORES // 2
EPT = N_EDGES // 16
NECHUNK = EPT // CHUNK
TRASH_SH = HALF + 8

_NEG_INF = float("-inf")


def _prefix16(x, lane):
    for k in (1, 2, 4, 8):
        idx = jnp.maximum(lane - k, 0)
        x = x + x.at[idx].get(mode="promise_in_bounds") * (lane >= k).astype(jnp.int32)
    return x


def _rsqrt16(x):
    i = plsc.bitcast(x, jnp.int32)
    i = 0x5F3759DF - lax.shift_right_logical(i, 1)
    y = plsc.bitcast(i, jnp.float32)
    for _ in range(4):
        y = y * (1.5 - 0.5 * x * y * y)
    return y


def _stage1_body(alloc_hbm, qe_hbm, dummy_hbm, dst_hbm, w_hbm,
                 out_hbm, dinv_hbm,
                 acc, rows, rows_b, achunk, wchunk, qidx, dloc, wlist, dinv_buf,
                 counts, deg_s, dummy_v, sem, sem_b):
    cid = lax.axis_index("c")
    sid = lax.axis_index("s")
    wid = sid * 2 + cid
    base = pl.multiple_of(wid * CPT, 8)
    npc = jnp.minimum(CPT, N_CORES - base)

    pltpu.sync_copy(dummy_hbm, dummy_v)

    def _zi(i, _):
        qidx[pl.ds(i * 16, 16)] = jnp.zeros((16,), jnp.int32)
        return 0
    lax.fori_loop(0, (CHUNK + 2 * GROWS) // 16, _zi, 0)

    trash16 = jnp.full((16,), CPT, jnp.int32)
    neg16 = jnp.full((16,), _NEG_INF, jnp.float32)
    lane = lax.iota(jnp.int32, 16)

    def _dz(i, _):
        deg_s[i] = jnp.float32(1.0)
        return 0
    lax.fori_loop(0, CPT + 1, _dz, 0)

    def _dchunk(ch, _):
        pos = ch * CHUNK
        pltpu.sync_copy(dst_hbm.at[pl.ds(pos, CHUNK)], achunk)
        pltpu.sync_copy(w_hbm.at[pl.ds(pos, CHUNK)], wchunk)

        def _rt(i, _):
            dloc[pl.ds(i * 16, 16)] = trash16
            return 0
        lax.fori_loop(0, (CHUNK + 16) // 16, _rt, 0)

        def _dscan(i, nq):
            dv = achunk[pl.ds(i * 16, 16)]
            wv = wchunk[pl.ds(i * 16, 16)]
            m = (dv >= base) & (dv < base + npc)
            mi = m.astype(jnp.int32)
            incl = _prefix16(mi, lane)
            dest = nq + incl - mi
            plsc.store_scatter(dloc, [dest], dv - base, mask=m)
            plsc.store_scatter(wlist, [dest], wv, mask=m)
            return nq + incl[15]
        nq = lax.fori_loop(0, NSCAN, _dscan, jnp.int32(0))

        def _acond(g):
            return g < nq

        def _abody(g):
            ga = pl.multiple_of(g, 16)
            cv = dloc[pl.ds(ga, 16)]
            wv = wlist[pl.ds(ga, 16)]
            for j in range(16):
                c = cv[j]
                deg_s[c] = deg_s[c] + wv[j]
            return g + 16
        lax.while_loop(_acond, _abody, jnp.int32(0))
        return 0
    lax.fori_loop(0, NECHUNK_ALL, _dchunk, 0)

    def _dv(r, _):
        for j in range(16):
            c = r * 16 + j
            sval = lax.broadcast_in_dim(deg_s[c], (16,), ())
            cvec = jnp.full((16,), c, jnp.int32)
            plsc.store_scatter(dinv_buf, [cvec], sval, mask=lane == j)
        x = dinv_buf[pl.ds(r * 16, 16)]
        dinv_buf[pl.ds(r * 16, 16)] = _rsqrt16(x)
        return 0
    lax.fori_loop(0, CPT // 16, _dv, 0)
    pltpu.sync_copy(dinv_buf.at[pl.ds(0, CPT)], dinv_hbm.at[pl.ds(base, CPT)])

    def _batch(b, _):
        def _ra(i, _):
            c = i // 16
            k = i % 16
            acc[c, pl.ds(k * 16, 16)] = neg16
            return 0
        lax.fori_loop(0, (CPT + 1) * 16, _ra, 0)

        def _rc(i, _):
            counts[i] = jnp.int32(0)
            return 0
        lax.fori_loop(0, CPT + 1, _rc, 0)

        def _chunk(ch, _):
            pos = ch * CHUNK
            pltpu.sync_copy(alloc_hbm.at[pl.ds(b * N_QUBITS + pos, CHUNK)], achunk)

            def _rt(i, _):
                dloc[pl.ds(i * 16, 16)] = trash16
                return 0
            lax.fori_loop(0, (CHUNK + 2 * GROWS) // 16, _rt, 0)

            def _scan(i, nq):
                v = achunk[pl.ds(i * 16, 16)]
                m = (v >= base) & (v < base + npc)
                mi = m.astype(jnp.int32)
                incl = _prefix16(mi, lane)
                dest = nq + incl - mi
                plsc.store_scatter(qidx, [dest], pos + i * 16 + lane, mask=m)
                plsc.store_scatter(dloc, [dest], v - base, mask=m)
                return nq + incl[15]
            nq = lax.fori_loop(0, NSCAN, _scan, jnp.int32(0))

            def _gcond(g):
                return g < nq

            def _gbody(g):
                ga = pl.multiple_of(g, GROWS)
                pltpu.async_copy(qe_hbm.at[qidx.at[pl.ds(ga, GROWS)]], rows,
                                 sem).wait()
                for g2 in range(0, GROWS, 16):
                    cv = dloc[pl.ds(pl.multiple_of(g + g2, 16), 16)]
                    for j in range(16):
                        c = cv[j]
                        news = [jnp.maximum(acc[c, pl.ds(k * 16, 16)],
                                            rows[g2 + j, pl.ds(k * 16, 16)])
                                for k in range(16)]
                        for k in range(16):
                            acc[c, pl.ds(k * 16, 16)] = news[k]
                        counts[c] = counts[c] + 1
                return g + GROWS
            lax.while_loop(_gcond, _gbody, jnp.int32(0))
            return 0
        lax.fori_loop(0, NCHUNK, _chunk, 0)

        def _fin(c, _):
            madd = jnp.where(counts[c] < CORE_CAP, jnp.float32(0.0),
                             jnp.float32(_NEG_INF))
            mv = lax.broadcast_in_dim(madd, (16,), ())
            for k in range(16):
                sl = pl.ds(k * 16, 16)
                acc[c, sl] = jnp.maximum(acc[c, sl], dummy_v[sl] + mv)
            return 0
        lax.fori_loop(0, npc, _fin, 0)

        @pl.when(wid < NW - 1)
        def _():
            pltpu.sync_copy(acc.at[pl.ds(0, CPT)], out_hbm.at[b, pl.ds(base, CPT)])

        @pl.when(wid == NW - 1)
        def _():
            last = N_CORES - (NW - 1) * CPT
            pltpu.sync_copy(acc.at[pl.ds(0, last)], out_hbm.at[b, pl.ds(base, last)])
        return 0

    lax.fori_loop(0, B, _batch, 0)


_STAGE1_SCRATCH = [
    pltpu.VMEM((CPT + 1, D), jnp.float32),
    pltpu.VMEM((GROWS, D), jnp.float32),
    pltpu.VMEM((GROWS, D), jnp.float32),
    pltpu.VMEM((CHUNK,), jnp.int32),
    pltpu.VMEM((CHUNK,), jnp.float32),
    pltpu.VMEM((CHUNK + 2 * GROWS,), jnp.int32),
    pltpu.VMEM((CHUNK + 2 * GROWS,), jnp.int32),
    pltpu.VMEM((CHUNK + 16,), jnp.float32),
    pltpu.VMEM((CPT + 16,), jnp.float32),
    pltpu.SMEM((CPT + 1,), jnp.int32),
    pltpu.SMEM((CPT + 1,), jnp.float32),
    pltpu.VMEM((D,), jnp.float32),
    pltpu.SemaphoreType.DMA,
    pltpu.SemaphoreType.DMA,
]

_SC_MESH = plsc.VectorSubcoreMesh(core_axis_name="c", subcore_axis_name="s",
                                  num_cores=2, num_subcores=16)

_stage1 = functools.partial(
    pl.kernel,
    out_type=(jax.ShapeDtypeStruct((B, N_CORES, D), jnp.float32),
              jax.ShapeDtypeStruct((NW * CPT,), jnp.float32)),
    mesh=_SC_MESH,
    compiler_params=pltpu.CompilerParams(needs_layout_passes=False),
    scratch_types=_STAGE1_SCRATCH,
)(_stage1_body)


LB = 2048
CAP = 162688


def _route_body(src_hbm, dst_hbm, w_hbm, dinv_hbm,
                es_hbm, ec_hbm, en_hbm, ecnt_hbm,
                dinv_v, schunk, dchunk, wchunk, sidx, nrm, dloc, cnt_v, sem):
    cid = lax.axis_index("c")
    sid = lax.axis_index("s")
    wid = sid * 2 + cid
    base = pl.multiple_of(wid * CPT, 8)
    npc = jnp.minimum(CPT, N_CORES - base)
    lane = lax.iota(jnp.int32, 16)
    trash16 = jnp.full((16,), CPT, jnp.int32)

    pltpu.sync_copy(dinv_hbm, dinv_v)

    def _zi(i, _):
        sidx[pl.ds(i * 16, 16)] = jnp.zeros((16,), jnp.int32)
        nrm[pl.ds(i * 16, 16)] = jnp.zeros((16,), jnp.float32)
        return 0
    lax.fori_loop(0, LB // 16, _zi, 0)

    def _chunk(ch, off):
        epos = ch * CHUNK
        pltpu.sync_copy(src_hbm.at[pl.ds(epos, CHUNK)], schunk)
        pltpu.sync_copy(dst_hbm.at[pl.ds(epos, CHUNK)], dchunk)
        pltpu.sync_copy(w_hbm.at[pl.ds(epos, CHUNK)], wchunk)

        def _rt(i, _):
            dloc[pl.ds(i * 16, 16)] = trash16
            return 0
        lax.fori_loop(0, LB // 16, _rt, 0)

        def _scan(i, nq):
            sv = schunk[pl.ds(i * 16, 16)]
            dv = dchunk[pl.ds(i * 16, 16)]
            wv = wchunk[pl.ds(i * 16, 16)]
            m = (dv >= base) & (dv < base + npc)
            mi = m.astype(jnp.int32)
            incl = _prefix16(mi, lane)
            dest = nq + incl - mi
            nv = plsc.load_gather(dinv_v, [sv]) * wv * plsc.load_gather(dinv_v, [dv])
            plsc.store_scatter(sidx, [dest], sv, mask=m)
            plsc.store_scatter(nrm, [dest], nv, mask=m)
            plsc.store_scatter(dloc, [dest], dv - base, mask=m)
            return nq + incl[15]
        nq = lax.fori_loop(0, NSCAN, _scan, jnp.int32(0))

        offa = pl.multiple_of(off, 8)
        pltpu.sync_copy(sidx, es_hbm.at[pl.ds(wid * CAP + offa, LB)])
        pltpu.sync_copy(dloc, ec_hbm.at[pl.ds(wid * CAP + offa, LB)])
        pltpu.sync_copy(nrm, en_hbm.at[pl.ds(wid * CAP + offa, LB)])
        return off + ((nq + 7) // 8) * 8
    off = lax.fori_loop(0, NECHUNK_ALL, _chunk, jnp.int32(0))

    def _tb(i, _):
        sidx[pl.ds(i * 16, 16)] = jnp.zeros((16,), jnp.int32)
        nrm[pl.ds(i * 16, 16)] = jnp.zeros((16,), jnp.float32)
        dloc[pl.ds(i * 16, 16)] = trash16
        return 0
    lax.fori_loop(0, LB // 16, _tb, 0)
    offa = pl.multiple_of(off, 8)
    pltpu.sync_copy(sidx, es_hbm.at[pl.ds(wid * CAP + offa, LB)])
    pltpu.sync_copy(dloc, ec_hbm.at[pl.ds(wid * CAP + offa, LB)])
    pltpu.sync_copy(nrm, en_hbm.at[pl.ds(wid * CAP + offa, LB)])

    cnt_v[pl.ds(0, 16)] = lax.broadcast_in_dim(off, (16,), ())
    pltpu.sync_copy(cnt_v, ecnt_hbm.at[pl.ds(wid * 16, 16)])


_ROUTE_SCRATCH = [
    pltpu.VMEM((NW * CPT,), jnp.float32),
    pltpu.VMEM((CHUNK,), jnp.int32),
    pltpu.VMEM((CHUNK,), jnp.int32),
    pltpu.VMEM((CHUNK,), jnp.float32),
    pltpu.VMEM((LB,), jnp.int32),
    pltpu.VMEM((LB,), jnp.float32),
    pltpu.VMEM((LB,), jnp.int32),
    pltpu.VMEM((16,), jnp.int32),
    pltpu.SemaphoreType.DMA,
]

_route = functools.partial(
    pl.kernel,
    out_type=(jax.ShapeDtypeStruct((NW * CAP,), jnp.int32),
              jax.ShapeDtypeStruct((NW * CAP,), jnp.int32),
              jax.ShapeDtypeStruct((NW * CAP,), jnp.float32),
              jax.ShapeDtypeStruct((NW * 16,), jnp.int32)),
    mesh=_SC_MESH,
    compiler_params=pltpu.CompilerParams(needs_layout_passes=False),
    scratch_types=_ROUTE_SCRATCH,
)(_route_body)


def _msg_body(h_hbm, es_hbm, ec_hbm, en_hbm, ecnt_hbm, dinv_hbm, bias_hbm,
              y_hbm,
              dinv_v, lsrc, lcore, lnrm, gidx2, acc, rows2,
              hrows, bias_v, cnt_v, sem, sem_b):
    cid = lax.axis_index("c")
    sid = lax.axis_index("s")
    wid = sid * 2 + cid
    base = pl.multiple_of(wid * CPT, 8)
    npc = jnp.minimum(CPT, N_CORES - base)
    lane = lax.iota(jnp.int32, 16)
    zero16 = jnp.zeros((16,), jnp.float32)

    pltpu.sync_copy(dinv_hbm, dinv_v)
    pltpu.sync_copy(bias_hbm, bias_v)
    pltpu.sync_copy(ecnt_hbm.at[pl.ds(wid * 16, 16)], cnt_v)
    ntot = cnt_v[pl.ds(0, 16)][0]

    def _batch(b, _):
        bN = b * N_CORES

        def _za(i, _):
            acc[i // 16, pl.ds((i % 16) * 16, 16)] = zero16
            return 0
        lax.fori_loop(0, (CPT + 1) * 16, _za, 0)

        def _bcond(o):
            return o < ntot

        def _bbody(o):
            oa = pl.multiple_of(o, 8)
            pltpu.sync_copy(es_hbm.at[pl.ds(wid * CAP + oa, LB)],
                            lsrc.at[pl.ds(0, LB)])
            pltpu.sync_copy(ec_hbm.at[pl.ds(wid * CAP + oa, LB)], lcore)
            pltpu.sync_copy(en_hbm.at[pl.ds(wid * CAP + oa, LB)], lnrm)
            ng = (jnp.minimum(LB, ntot - oa) + GROWS - 1) // GROWS

            def _fill(p, ga):
                gidx2[p, pl.ds(0, 16)] = lsrc[pl.ds(pl.multiple_of(ga, 16), 16)] + bN
                gidx2[p, pl.ds(16, 16)] = lsrc[pl.ds(pl.multiple_of(ga + 16, 16), 16)] + bN

            def _issue(p):
                @pl.when(p == 0)
                def _():
                    pltpu.async_copy(h_hbm.at[gidx2.at[0]], rows2.at[0], sem)

                @pl.when(p == 1)
                def _():
                    pltpu.async_copy(h_hbm.at[gidx2.at[1]], rows2.at[1], sem_b)

            _fill(jnp.int32(0), jnp.int32(0))
            _issue(jnp.int32(0))

            def _group(gi, _):
                p = lax.rem(gi, 2)
                pn = 1 - p

                @pl.when(gi + 1 < ng)
                def _():
                    _fill(pn, (gi + 1) * GROWS)
                    _issue(pn)

                @pl.when(p == 0)
                def _():
                    pltpu.make_async_copy(h_hbm.at[gidx2.at[0]], rows2.at[0],
                                          sem).wait()

                @pl.when(p == 1)
                def _():
                    pltpu.make_async_copy(h_hbm.at[gidx2.at[1]], rows2.at[1],
                                          sem_b).wait()

                ga = gi * GROWS
                for g2 in range(0, GROWS, 16):
                    ga2 = pl.multiple_of(ga + g2, 16)
                    cv = lcore[pl.ds(ga2, 16)]
                    nvv = lnrm[pl.ds(ga2, 16)]
                    for j in range(16):
                        cvec = lax.broadcast_in_dim(cv[j], (16,), ())
                        sv16 = lax.broadcast_in_dim(nvv[j], (16,), ())
                        for k in range(16):
                            sl = pl.ds(k * 16, 16)
                            plsc.addupdate_scatter(
                                acc, [cvec, k * 16 + lane],
                                rows2[p, g2 + j, sl] * sv16)
                return 0
            lax.fori_loop(0, ng, _group, 0)
            return o + LB
        lax.while_loop(_bcond, _bbody, jnp.int32(0))

        def _f(i, _):
            row0 = pl.multiple_of(i * 8, 8)
            grow0 = pl.multiple_of(bN + base + row0, 8)
            pltpu.sync_copy(h_hbm.at[pl.ds(grow0, 8)], hrows)
            dinvv = dinv_v[pl.ds(pl.multiple_of(base + row0, 8), 16)]
            for rr in range(8):
                ns = dinvv[rr]
                nsv = lax.broadcast_in_dim(ns * ns, (16,), ())
                for k in range(16):
                    sl = pl.ds(k * 16, 16)
                    y = acc[row0 + rr, sl] + hrows[rr, sl] * nsv + bias_v[sl]
                    acc[row0 + rr, sl] = jnp.maximum(y, 0.0)
            return 0
        lax.fori_loop(0, npc // 8, _f, 0)

        @pl.when(wid < NW - 1)
        def _():
            pltpu.sync_copy(acc.at[pl.ds(0, CPT)],
                            y_hbm.at[pl.ds(bN + base, CPT)])

        @pl.when(wid == NW - 1)
        def _():
            last = N_CORES - (NW - 1) * CPT
            pltpu.sync_copy(acc.at[pl.ds(0, last)],
                            y_hbm.at[pl.ds(bN + base, last)])
        return 0

    lax.fori_loop(0, B, _batch, 0)


_MSG_SCRATCH = [
    pltpu.VMEM((NW * CPT,), jnp.float32),
    pltpu.VMEM((LB + GROWS,), jnp.int32),
    pltpu.VMEM((LB,), jnp.int32),
    pltpu.VMEM((LB,), jnp.float32),
    pltpu.VMEM((2, GROWS), jnp.int32),
    pltpu.VMEM((CPT + 1, D), jnp.float32),
    pltpu.VMEM((2, GROWS, D), jnp.float32),
    pltpu.VMEM((8, D), jnp.float32),
    pltpu.VMEM((D,), jnp.float32),
    pltpu.VMEM((16,), jnp.int32),
    pltpu.SemaphoreType.DMA,
    pltpu.SemaphoreType.DMA,
]

_msg = functools.partial(
    pl.kernel,
    out_type=jax.ShapeDtypeStruct((B * N_CORES, D), jnp.float32),
    mesh=_SC_MESH,
    compiler_params=pltpu.CompilerParams(needs_layout_passes=False),
    scratch_types=_MSG_SCRATCH,
)(_msg_body)


def _mm_body(x_ref, w_ref, o_ref):
    o_ref[...] = jnp.dot(x_ref[...], w_ref[...], preferred_element_type=jnp.float32)


def _matmul(x, w):
    rows = x.shape[0]
    blk = 400
    return pl.pallas_call(
        _mm_body,
        grid=(rows // blk,),
        in_specs=[
            pl.BlockSpec((blk, D), lambda i: (i, 0)),
            pl.BlockSpec((D, D), lambda i: (0, 0)),
        ],
        out_specs=pl.BlockSpec((blk, D), lambda i: (i, 0)),
        out_shape=jax.ShapeDtypeStruct((rows, D), jnp.float32),
    )(x, w)


def kernel(core_allocs, qubit_embs, dummy_qubit_emb, edge_index, edge_weight, W1, b1, W2, b2):
    src = edge_index[0]
    dst = edge_index[1]
    pre_embs, dinv = _stage1(core_allocs.reshape(-1), qubit_embs,
                             dummy_qubit_emb, dst, edge_weight)
    es, ec, en, ecnt = _route(src, dst, edge_weight, dinv)
    h1 = _matmul(pre_embs.reshape(B * N_CORES, D), W1)
    x1 = _msg(h1, es, ec, en, ecnt, dinv, b1)
    h2 = _matmul(x1, W2)
    x2 = _msg(h2, es, ec, en, ecnt, dinv, b2)
    return x2.reshape(B, N_CORES, D)


# speedup vs baseline: 1.5953x; 1.0192x over previous
import functools
import jax
import jax.numpy as jnp
from jax import lax
from jax.experimental import pallas as pl
from jax.experimental.pallas import tpu as pltpu
from jax.experimental.pallas import tpu_sc as plsc

N_CORES = 10000
N_QUBITS = 100000
CORE_CAP = 32
N_EDGES = 160000
D = 256
B = 4

NW = 32
CPT = 320
CHUNK = 2000
NSCAN = CHUNK // 16
NCHUNK = N_QUBITS // CHUNK
NECHUNK_ALL = N_EDGES // CHUNK
GROWS = 32

HALF = N_CORES // 2
EPT = N_EDGES // 16
NECHUNK = EPT // CHUNK
TRASH_SH = HALF + 8

_NEG_INF = float("-inf")


def _prefix16(x, lane):
    for k in (1, 2, 4, 8):
        idx = jnp.maximum(lane - k, 0)
        x = x + x.at[idx].get(mode="promise_in_bounds") * (lane >= k).astype(jnp.int32)
    return x


def _rsqrt16(x):
    i = plsc.bitcast(x, jnp.int32)
    i = 0x5F3759DF - lax.shift_right_logical(i, 1)
    y = plsc.bitcast(i, jnp.float32)
    for _ in range(4):
        y = y * (1.5 - 0.5 * x * y * y)
    return y


def _stage1_body(alloc_hbm, qe_hbm, dummy_hbm, dst_hbm, w_hbm,
                 out_hbm, dinv_hbm,
                 acc, rows, achunk, wchunk, qidx, dloc, wlist, dinv_buf,
                 counts, deg_s, dummy_v, sem, sem_b):
    cid = lax.axis_index("c")
    sid = lax.axis_index("s")
    wid = sid * 2 + cid
    base = pl.multiple_of(wid * CPT, 8)
    npc = jnp.minimum(CPT, N_CORES - base)

    pltpu.sync_copy(dummy_hbm, dummy_v)

    def _zi(i, _):
        qidx[pl.ds(i * 16, 16)] = jnp.zeros((16,), jnp.int32)
        return 0
    lax.fori_loop(0, (CHUNK + 2 * GROWS) // 16, _zi, 0)

    trash16 = jnp.full((16,), CPT, jnp.int32)
    neg16 = jnp.full((16,), _NEG_INF, jnp.float32)
    lane = lax.iota(jnp.int32, 16)

    def _dz(i, _):
        deg_s[i] = jnp.float32(1.0)
        return 0
    lax.fori_loop(0, CPT + 1, _dz, 0)

    def _dchunk(ch, _):
        pos = ch * CHUNK
        pltpu.sync_copy(dst_hbm.at[pl.ds(pos, CHUNK)], achunk)
        pltpu.sync_copy(w_hbm.at[pl.ds(pos, CHUNK)], wchunk)

        def _rt(i, _):
            dloc[pl.ds(i * 16, 16)] = trash16
            return 0
        lax.fori_loop(0, (CHUNK + 16) // 16, _rt, 0)

        def _dscan(i, nq):
            dv = achunk[pl.ds(i * 16, 16)]
            wv = wchunk[pl.ds(i * 16, 16)]
            m = (dv >= base) & (dv < base + npc)
            mi = m.astype(jnp.int32)
            incl = _prefix16(mi, lane)
            dest = nq + incl - mi
            plsc.store_scatter(dloc, [dest], dv - base, mask=m)
            plsc.store_scatter(wlist, [dest], wv, mask=m)
            return nq + incl[15]
        nq = lax.fori_loop(0, NSCAN, _dscan, jnp.int32(0))

        def _acond(g):
            return g < nq

        def _abody(g):
            ga = pl.multiple_of(g, 16)
            cv = dloc[pl.ds(ga, 16)]
            wv = wlist[pl.ds(ga, 16)]
            for j in range(16):
                c = cv[j]
                deg_s[c] = deg_s[c] + wv[j]
            return g + 16
        lax.while_loop(_acond, _abody, jnp.int32(0))
        return 0
    lax.fori_loop(0, NECHUNK_ALL, _dchunk, 0)

    def _dv(r, _):
        for j in range(16):
            c = r * 16 + j
            sval = lax.broadcast_in_dim(deg_s[c], (16,), ())
            cvec = jnp.full((16,), c, jnp.int32)
            plsc.store_scatter(dinv_buf, [cvec], sval, mask=lane == j)
        x = dinv_buf[pl.ds(r * 16, 16)]
        dinv_buf[pl.ds(r * 16, 16)] = _rsqrt16(x)
        return 0
    lax.fori_loop(0, CPT // 16, _dv, 0)
    pltpu.sync_copy(dinv_buf.at[pl.ds(0, CPT)], dinv_hbm.at[pl.ds(base, CPT)])

    def _batch(b, _):
        def _ra(i, _):
            c = i // 16
            k = i % 16
            acc[c, pl.ds(k * 16, 16)] = neg16
            return 0
        lax.fori_loop(0, (CPT + 1) * 16, _ra, 0)

        def _rc(i, _):
            counts[i] = jnp.int32(0)
            return 0
        lax.fori_loop(0, CPT + 1, _rc, 0)

        def _chunk(ch, _):
            pos = ch * CHUNK
            pltpu.sync_copy(alloc_hbm.at[pl.ds(b * N_QUBITS + pos, CHUNK)], achunk)

            def _rt(i, _):
                dloc[pl.ds(i * 16, 16)] = trash16
                return 0
            lax.fori_loop(0, (CHUNK + 2 * GROWS) // 16, _rt, 0)

            def _scan(i, nq):
                v = achunk[pl.ds(i * 16, 16)]
                m = (v >= base) & (v < base + npc)
                mi = m.astype(jnp.int32)
                incl = _prefix16(mi, lane)
                dest = nq + incl - mi
                plsc.store_scatter(qidx, [dest], pos + i * 16 + lane, mask=m)
                plsc.store_scatter(dloc, [dest], v - base, mask=m)
                return nq + incl[15]
            nq = lax.fori_loop(0, NSCAN, _scan, jnp.int32(0))

            ngq = (nq + GROWS - 1) // GROWS

            def _issue(p, ga):
                @pl.when(p == 0)
                def _():
                    pltpu.async_copy(qe_hbm.at[qidx.at[pl.ds(ga, GROWS)]],
                                     rows.at[0], sem)

                @pl.when(p == 1)
                def _():
                    pltpu.async_copy(qe_hbm.at[qidx.at[pl.ds(ga, GROWS)]],
                                     rows.at[1], sem_b)

            @pl.when(ngq > 0)
            def _():
                _issue(jnp.int32(0), pl.multiple_of(jnp.int32(0), GROWS))

            def _gbody(gi, _):
                p = lax.rem(gi, 2)
                pn = 1 - p

                @pl.when(gi + 1 < ngq)
                def _():
                    _issue(pn, pl.multiple_of((gi + 1) * GROWS, GROWS))

                @pl.when(p == 0)
                def _():
                    pltpu.make_async_copy(qe_hbm.at[qidx.at[pl.ds(0, GROWS)]],
                                          rows.at[0], sem).wait()

                @pl.when(p == 1)
                def _():
                    pltpu.make_async_copy(qe_hbm.at[qidx.at[pl.ds(0, GROWS)]],
                                          rows.at[1], sem_b).wait()

                ga = gi * GROWS
                for g2 in range(0, GROWS, 16):
                    cv = dloc[pl.ds(pl.multiple_of(ga + g2, 16), 16)]
                    for j in range(16):
                        c = cv[j]
                        news = [jnp.maximum(acc[c, pl.ds(k * 16, 16)],
                                            rows[p, g2 + j, pl.ds(k * 16, 16)])
                                for k in range(16)]
                        for k in range(16):
                            acc[c, pl.ds(k * 16, 16)] = news[k]
                        counts[c] = counts[c] + 1
                return 0
            lax.fori_loop(0, ngq, _gbody, 0)
            return 0
        lax.fori_loop(0, NCHUNK, _chunk, 0)

        def _fin(c, _):
            madd = jnp.where(counts[c] < CORE_CAP, jnp.float32(0.0),
                             jnp.float32(_NEG_INF))
            mv = lax.broadcast_in_dim(madd, (16,), ())
            for k in range(16):
                sl = pl.ds(k * 16, 16)
                acc[c, sl] = jnp.maximum(acc[c, sl], dummy_v[sl] + mv)
            return 0
        lax.fori_loop(0, npc, _fin, 0)

        @pl.when(wid < NW - 1)
        def _():
            pltpu.sync_copy(acc.at[pl.ds(0, CPT)], out_hbm.at[b, pl.ds(base, CPT)])

        @pl.when(wid == NW - 1)
        def _():
            last = N_CORES - (NW - 1) * CPT
            pltpu.sync_copy(acc.at[pl.ds(0, last)], out_hbm.at[b, pl.ds(base, last)])
        return 0

    lax.fori_loop(0, B, _batch, 0)


_STAGE1_SCRATCH = [
    pltpu.VMEM((CPT + 1, D), jnp.float32),
    pltpu.VMEM((2, GROWS, D), jnp.float32),
    pltpu.VMEM((CHUNK,), jnp.int32),
    pltpu.VMEM((CHUNK,), jnp.float32),
    pltpu.VMEM((CHUNK + 2 * GROWS,), jnp.int32),
    pltpu.VMEM((CHUNK + 2 * GROWS,), jnp.int32),
    pltpu.VMEM((CHUNK + 16,), jnp.float32),
    pltpu.VMEM((CPT + 16,), jnp.float32),
    pltpu.SMEM((CPT + 1,), jnp.int32),
    pltpu.SMEM((CPT + 1,), jnp.float32),
    pltpu.VMEM((D,), jnp.float32),
    pltpu.SemaphoreType.DMA,
    pltpu.SemaphoreType.DMA,
]

_SC_MESH = plsc.VectorSubcoreMesh(core_axis_name="c", subcore_axis_name="s",
                                  num_cores=2, num_subcores=16)

_stage1 = functools.partial(
    pl.kernel,
    out_type=(jax.ShapeDtypeStruct((B, N_CORES, D), jnp.float32),
              jax.ShapeDtypeStruct((NW * CPT,), jnp.float32)),
    mesh=_SC_MESH,
    compiler_params=pltpu.CompilerParams(needs_layout_passes=False),
    scratch_types=_STAGE1_SCRATCH,
)(_stage1_body)


LB = 2048
CAP = 162688


def _route_body(src_hbm, dst_hbm, w_hbm, dinv_hbm,
                es_hbm, ec_hbm, en_hbm, ecnt_hbm,
                dinv_v, schunk, dchunk, wchunk, sidx, nrm, dloc, cnt_v, sem):
    cid = lax.axis_index("c")
    sid = lax.axis_index("s")
    wid = sid * 2 + cid
    base = pl.multiple_of(wid * CPT, 8)
    npc = jnp.minimum(CPT, N_CORES - base)
    lane = lax.iota(jnp.int32, 16)
    trash16 = jnp.full((16,), CPT, jnp.int32)

    pltpu.sync_copy(dinv_hbm, dinv_v)

    def _zi(i, _):
        sidx[pl.ds(i * 16, 16)] = jnp.zeros((16,), jnp.int32)
        nrm[pl.ds(i * 16, 16)] = jnp.zeros((16,), jnp.float32)
        return 0
    lax.fori_loop(0, LB // 16, _zi, 0)

    def _chunk(ch, off):
        epos = ch * CHUNK
        pltpu.sync_copy(src_hbm.at[pl.ds(epos, CHUNK)], schunk)
        pltpu.sync_copy(dst_hbm.at[pl.ds(epos, CHUNK)], dchunk)
        pltpu.sync_copy(w_hbm.at[pl.ds(epos, CHUNK)], wchunk)

        def _rt(i, _):
            dloc[pl.ds(i * 16, 16)] = trash16
            return 0
        lax.fori_loop(0, LB // 16, _rt, 0)

        def _scan(i, nq):
            sv = schunk[pl.ds(i * 16, 16)]
            dv = dchunk[pl.ds(i * 16, 16)]
            wv = wchunk[pl.ds(i * 16, 16)]
            m = (dv >= base) & (dv < base + npc)
            mi = m.astype(jnp.int32)
            incl = _prefix16(mi, lane)
            dest = nq + incl - mi
            nv = plsc.load_gather(dinv_v, [sv]) * wv * plsc.load_gather(dinv_v, [dv])
            plsc.store_scatter(sidx, [dest], sv, mask=m)
            plsc.store_scatter(nrm, [dest], nv, mask=m)
            plsc.store_scatter(dloc, [dest], dv - base, mask=m)
            return nq + incl[15]
        nq = lax.fori_loop(0, NSCAN, _scan, jnp.int32(0))

        offa = pl.multiple_of(off, 8)
        pltpu.sync_copy(sidx, es_hbm.at[pl.ds(wid * CAP + offa, LB)])
        pltpu.sync_copy(dloc, ec_hbm.at[pl.ds(wid * CAP + offa, LB)])
        pltpu.sync_copy(nrm, en_hbm.at[pl.ds(wid * CAP + offa, LB)])
        return off + ((nq + 7) // 8) * 8
    off = lax.fori_loop(0, NECHUNK_ALL, _chunk, jnp.int32(0))

    def _tb(i, _):
        sidx[pl.ds(i * 16, 16)] = jnp.zeros((16,), jnp.int32)
        nrm[pl.ds(i * 16, 16)] = jnp.zeros((16,), jnp.float32)
        dloc[pl.ds(i * 16, 16)] = trash16
        return 0
    lax.fori_loop(0, LB // 16, _tb, 0)
    offa = pl.multiple_of(off, 8)
    pltpu.sync_copy(sidx, es_hbm.at[pl.ds(wid * CAP + offa, LB)])
    pltpu.sync_copy(dloc, ec_hbm.at[pl.ds(wid * CAP + offa, LB)])
    pltpu.sync_copy(nrm, en_hbm.at[pl.ds(wid * CAP + offa, LB)])

    cnt_v[pl.ds(0, 16)] = lax.broadcast_in_dim(off, (16,), ())
    pltpu.sync_copy(cnt_v, ecnt_hbm.at[pl.ds(wid * 16, 16)])


_ROUTE_SCRATCH = [
    pltpu.VMEM((NW * CPT,), jnp.float32),
    pltpu.VMEM((CHUNK,), jnp.int32),
    pltpu.VMEM((CHUNK,), jnp.int32),
    pltpu.VMEM((CHUNK,), jnp.float32),
    pltpu.VMEM((LB,), jnp.int32),
    pltpu.VMEM((LB,), jnp.float32),
    pltpu.VMEM((LB,), jnp.int32),
    pltpu.VMEM((16,), jnp.int32),
    pltpu.SemaphoreType.DMA,
]

_route = functools.partial(
    pl.kernel,
    out_type=(jax.ShapeDtypeStruct((NW * CAP,), jnp.int32),
              jax.ShapeDtypeStruct((NW * CAP,), jnp.int32),
              jax.ShapeDtypeStruct((NW * CAP,), jnp.float32),
              jax.ShapeDtypeStruct((NW * 16,), jnp.int32)),
    mesh=_SC_MESH,
    compiler_params=pltpu.CompilerParams(needs_layout_passes=False),
    scratch_types=_ROUTE_SCRATCH,
)(_route_body)


def _msg_body(h_hbm, es_hbm, ec_hbm, en_hbm, ecnt_hbm, dinv_hbm, bias_hbm,
              y_hbm,
              dinv_v, lsrc, lcore, lnrm, gidx2, acc, rows2,
              hrows, bias_v, cnt_v, sem, sem_b):
    cid = lax.axis_index("c")
    sid = lax.axis_index("s")
    wid = sid * 2 + cid
    base = pl.multiple_of(wid * CPT, 8)
    npc = jnp.minimum(CPT, N_CORES - base)
    lane = lax.iota(jnp.int32, 16)
    zero16 = jnp.zeros((16,), jnp.float32)

    pltpu.sync_copy(dinv_hbm, dinv_v)
    pltpu.sync_copy(bias_hbm, bias_v)
    pltpu.sync_copy(ecnt_hbm.at[pl.ds(wid * 16, 16)], cnt_v)
    ntot = cnt_v[pl.ds(0, 16)][0]

    def _batch(b, _):
        bN = b * N_CORES

        def _za(i, _):
            acc[i // 16, pl.ds((i % 16) * 16, 16)] = zero16
            return 0
        lax.fori_loop(0, (CPT + 1) * 16, _za, 0)

        def _bcond(o):
            return o < ntot

        def _bbody(o):
            oa = pl.multiple_of(o, 8)
            pltpu.sync_copy(es_hbm.at[pl.ds(wid * CAP + oa, LB)],
                            lsrc.at[pl.ds(0, LB)])
            pltpu.sync_copy(ec_hbm.at[pl.ds(wid * CAP + oa, LB)], lcore)
            pltpu.sync_copy(en_hbm.at[pl.ds(wid * CAP + oa, LB)], lnrm)
            ng = (jnp.minimum(LB, ntot - oa) + GROWS - 1) // GROWS

            def _fill(p, ga):
                gidx2[p, pl.ds(0, 16)] = lsrc[pl.ds(pl.multiple_of(ga, 16), 16)] + bN
                gidx2[p, pl.ds(16, 16)] = lsrc[pl.ds(pl.multiple_of(ga + 16, 16), 16)] + bN

            def _issue(p):
                @pl.when(p == 0)
                def _():
                    pltpu.async_copy(h_hbm.at[gidx2.at[0]], rows2.at[0], sem)

                @pl.when(p == 1)
                def _():
                    pltpu.async_copy(h_hbm.at[gidx2.at[1]], rows2.at[1], sem_b)

            _fill(jnp.int32(0), jnp.int32(0))
            _issue(jnp.int32(0))

            def _group(gi, _):
                p = lax.rem(gi, 2)
                pn = 1 - p

                @pl.when(gi + 1 < ng)
                def _():
                    _fill(pn, (gi + 1) * GROWS)
                    _issue(pn)

                @pl.when(p == 0)
                def _():
                    pltpu.make_async_copy(h_hbm.at[gidx2.at[0]], rows2.at[0],
                                          sem).wait()

                @pl.when(p == 1)
                def _():
                    pltpu.make_async_copy(h_hbm.at[gidx2.at[1]], rows2.at[1],
                                          sem_b).wait()

                ga = gi * GROWS
                for g2 in range(0, GROWS, 16):
                    ga2 = pl.multiple_of(ga + g2, 16)
                    cv = lcore[pl.ds(ga2, 16)]
                    nvv = lnrm[pl.ds(ga2, 16)]
                    for j in range(16):
                        cvec = lax.broadcast_in_dim(cv[j], (16,), ())
                        sv16 = lax.broadcast_in_dim(nvv[j], (16,), ())
                        for k in range(16):
                            sl = pl.ds(k * 16, 16)
                            plsc.addupdate_scatter(
                                acc, [cvec, k * 16 + lane],
                                rows2[p, g2 + j, sl] * sv16)
                return 0
            lax.fori_loop(0, ng, _group, 0)
            return o + LB
        lax.while_loop(_bcond, _bbody, jnp.int32(0))

        def _f(i, _):
            row0 = pl.multiple_of(i * 8, 8)
            grow0 = pl.multiple_of(bN + base + row0, 8)
            pltpu.sync_copy(h_hbm.at[pl.ds(grow0, 8)], hrows)
            dinvv = dinv_v[pl.ds(pl.multiple_of(base + row0, 8), 16)]
            for rr in range(8):
                ns = dinvv[rr]
                nsv = lax.broadcast_in_dim(ns * ns, (16,), ())
                for k in range(16):
                    sl = pl.ds(k * 16, 16)
                    y = acc[row0 + rr, sl] + hrows[rr, sl] * nsv + bias_v[sl]
                    acc[row0 + rr, sl] = jnp.maximum(y, 0.0)
            return 0
        lax.fori_loop(0, npc // 8, _f, 0)

        @pl.when(wid < NW - 1)
        def _():
            pltpu.sync_copy(acc.at[pl.ds(0, CPT)],
                            y_hbm.at[pl.ds(bN + base, CPT)])

        @pl.when(wid == NW - 1)
        def _():
            last = N_CORES - (NW - 1) * CPT
            pltpu.sync_copy(acc.at[pl.ds(0, last)],
                            y_hbm.at[pl.ds(bN + base, last)])
        return 0

    lax.fori_loop(0, B, _batch, 0)


_MSG_SCRATCH = [
    pltpu.VMEM((NW * CPT,), jnp.float32),
    pltpu.VMEM((LB + GROWS,), jnp.int32),
    pltpu.VMEM((LB,), jnp.int32),
    pltpu.VMEM((LB,), jnp.float32),
    pltpu.VMEM((2, GROWS), jnp.int32),
    pltpu.VMEM((CPT + 1, D), jnp.float32),
    pltpu.VMEM((2, GROWS, D), jnp.float32),
    pltpu.VMEM((8, D), jnp.float32),
    pltpu.VMEM((D,), jnp.float32),
    pltpu.VMEM((16,), jnp.int32),
    pltpu.SemaphoreType.DMA,
    pltpu.SemaphoreType.DMA,
]

_msg = functools.partial(
    pl.kernel,
    out_type=jax.ShapeDtypeStruct((B * N_CORES, D), jnp.float32),
    mesh=_SC_MESH,
    compiler_params=pltpu.CompilerParams(needs_layout_passes=False),
    scratch_types=_MSG_SCRATCH,
)(_msg_body)


def _mm_body(x_ref, w_ref, o_ref):
    o_ref[...] = jnp.dot(x_ref[...], w_ref[...], preferred_element_type=jnp.float32)


def _matmul(x, w):
    rows = x.shape[0]
    blk = 400
    return pl.pallas_call(
        _mm_body,
        grid=(rows // blk,),
        in_specs=[
            pl.BlockSpec((blk, D), lambda i: (i, 0)),
            pl.BlockSpec((D, D), lambda i: (0, 0)),
        ],
        out_specs=pl.BlockSpec((blk, D), lambda i: (i, 0)),
        out_shape=jax.ShapeDtypeStruct((rows, D), jnp.float32),
    )(x, w)


def kernel(core_allocs, qubit_embs, dummy_qubit_emb, edge_index, edge_weight, W1, b1, W2, b2):
    src = edge_index[0]
    dst = edge_index[1]
    pre_embs, dinv = _stage1(core_allocs.reshape(-1), qubit_embs,
                             dummy_qubit_emb, dst, edge_weight)
    es, ec, en, ecnt = _route(src, dst, edge_weight, dinv)
    h1 = _matmul(pre_embs.reshape(B * N_CORES, D), W1)
    x1 = _msg(h1, es, ec, en, ecnt, dinv, b1)
    h2 = _matmul(x1, W2)
    x2 = _msg(h2, es, ec, en, ecnt, dinv, b2)
    return x2.reshape(B, N_CORES, D)
